# Initial kernel scaffold; baseline (speedup 1.0000x reference)
import jax
import jax.numpy as jnp
from jax.experimental import pallas as pl


def kernel(hidden_states, sim_matrix, threshold, w1, w2):
    raise NotImplementedError("write your pallas kernel here")


# speedup vs baseline: 2.4085x; 2.4085x over previous
import jax
import jax.numpy as jnp
from jax.experimental import pallas as pl
from jax.experimental.pallas import tpu as pltpu


def _moe_step(x32_ref, sim_ref, thr_ref, w1_ref, w2_ref,
              final_ref, scores_ref, eof_ref, k_ref):
    e = pl.program_id(1)
    thr = thr_ref[0, 0]

    @pl.when(e == 0)
    def _():
        xf = x32_ref[...]
        xn = xf / (jnp.sqrt(jnp.sum(xf * xf, axis=1, keepdims=True)) + 1e-12)
        sm = sim_ref[...]
        wn = sm / (jnp.sqrt(jnp.sum(sm * sm, axis=0, keepdims=True)) + 1e-12)
        s = jnp.dot(xn, wn, preferred_element_type=jnp.float32)
        scores_ref[...] = s
        k_ref[...] = jnp.sum((s > thr).astype(jnp.int32), axis=1, keepdims=True)

    s_full = scores_ref[...]
    onehot = (jax.lax.broadcasted_iota(jnp.int32, s_full.shape, 1) == e)
    mask_col = jnp.sum(
        jnp.where((s_full > thr) & onehot, 1.0, 0.0), axis=1, keepdims=True)

    xb = x32_ref[...].astype(jnp.bfloat16)
    w1e = w1_ref[0]
    w2e = w2_ref[0]
    h = jax.lax.dot_general(xb, w1e, (((1,), (1,)), ((), ())),
                            preferred_element_type=jnp.float32)
    g = 0.5 * h * (1.0 + jax.lax.erf(h * 0.7071067811865476))
    out = jax.lax.dot_general(g.astype(jnp.bfloat16), w2e,
                              (((1,), (1,)), ((), ())),
                              preferred_element_type=jnp.float32)
    mo = out * mask_col
    eof_ref[:, e, :] = mo

    @pl.when(e == 0)
    def _():
        final_ref[...] = mo

    @pl.when(e != 0)
    def _():
        final_ref[...] += mo


def kernel(hidden_states, sim_matrix, threshold, w1, w2):
    Bb, Tt, Cc = hidden_states.shape
    Ee, Dff, _ = w1.shape
    N = Bb * Tt
    TB = 512
    NB = N // TB

    x32 = hidden_states.reshape(N, Cc)
    w1b = w1.astype(jnp.bfloat16)
    w2b = w2.astype(jnp.bfloat16)
    thr = threshold.reshape(1, 1)

    grid = (NB, Ee)
    out_shapes = (
        jax.ShapeDtypeStruct((N, Cc), jnp.float32),
        jax.ShapeDtypeStruct((N, Ee), jnp.float32),
        jax.ShapeDtypeStruct((N, Ee, Cc), jnp.float32),
        jax.ShapeDtypeStruct((N, 1), jnp.int32),
    )
    in_specs = [
        pl.BlockSpec((TB, Cc), lambda n, e: (n, 0)),
        pl.BlockSpec((Cc, Ee), lambda n, e: (0, 0)),
        pl.BlockSpec((1, 1), lambda n, e: (0, 0)),
        pl.BlockSpec((1, Dff, Cc), lambda n, e: (e, 0, 0)),
        pl.BlockSpec((1, Cc, Dff), lambda n, e: (e, 0, 0)),
    ]
    out_specs = (
        pl.BlockSpec((TB, Cc), lambda n, e: (n, 0)),
        pl.BlockSpec((TB, Ee), lambda n, e: (n, 0)),
        pl.BlockSpec((TB, Ee, Cc), lambda n, e: (n, 0, 0)),
        pl.BlockSpec((TB, 1), lambda n, e: (n, 0)),
    )
    final, scores, eof, k = pl.pallas_call(
        _moe_step,
        grid=grid,
        in_specs=in_specs,
        out_specs=out_specs,
        out_shape=out_shapes,
        compiler_params=pltpu.CompilerParams(
            dimension_semantics=("arbitrary", "arbitrary"),
            vmem_limit_bytes=63 * 1024 * 1024,
        ),
    )(x32, sim_matrix, thr, w1b, w2b)

    return (final.reshape(Bb, Tt, Cc), scores, eof, k.reshape(N))
